# baseline (speedup 1.0000x reference)
import functools

import jax
import jax.numpy as jnp
from jax import lax
from jax.experimental import pallas as pl
from jax.experimental.pallas import tpu as pltpu
from jax.experimental.pallas import tpu_sc as plsc

N = 10000
D = 128
E = 320000
NL = 6

NTILES = 32
CHUNK = 64
CPT = 160
EPT = CHUNK * CPT
EPAD = EPT * NTILES
WAUG = 144
ACC_ROWS = 10240
DUMMY_ROW = 10100
NPAD = ACC_ROWS
RPT = ACC_ROWS // 16
BLK = 32
NBLK = CPT // BLK
GP = CHUNK // 16

_f32 = jnp.float32
_i32 = jnp.int32


def _sc_body(h_hbm, ald_hbm, src_hbm, dst_hbm, ew_hbm, out_hbm,
             ald_v, srcb, dstb, ewb, hr_a, hr_b, acc, gsem, ssem):
    cid = lax.axis_index("c")
    sid = lax.axis_index("s")
    wid = cid * 16 + sid

    pltpu.sync_copy(ald_hbm, ald_v)

    zeros16 = jnp.zeros((16,), _f32)
    for r in range(CHUNK):
        for k in range(WAUG // 16):
            hr_a[r, pl.ds(16 * k, 16)] = zeros16
    for k in range(RPT // CHUNK):
        pltpu.sync_copy(hr_a, acc.at[pl.ds(sid * RPT + k * CHUNK, CHUNK)])
    plsc.subcore_barrier()

    col128 = jnp.full((16,), D, _i32)
    lane = lax.iota(_i32, 16)

    def _gather(j, hr):
        pltpu.async_copy(h_hbm.at[srcb.at[j]], hr, gsem)

    def _wait_gather():
        pltpu.make_async_copy(h_hbm.at[srcb.at[0]], hr_a, gsem).wait()

    def _scatter(j, hr):
        pltpu.async_copy(hr, acc.at[dstb.at[j]], ssem, add=True)

    def _drain_scatter():
        pltpu.make_async_copy(hr_a, acc.at[dstb.at[0]], ssem).wait()

    def _compute(j, hr):
        for g in range(GP):
            rows16 = 16 * g + lane
            d16 = dstb[j, pl.ds(16 * g, 16)]
            als16 = plsc.load_gather(hr, [rows16, col128])
            ald16 = plsc.load_gather(ald_v, [d16])
            a16 = als16 + ald16
            e16 = jnp.maximum(a16, 0.2 * a16)
            ex16 = jnp.exp(e16)
            w16 = ex16 * ewb[j, pl.ds(16 * g, 16)]
            plsc.store_scatter(hr, [rows16, col128], ex16)

            def _cols(ci, c2):
                for k in range(8):
                    cc = jnp.full((16,), 8 * ci + k, _i32)
                    v = plsc.load_gather(hr, [rows16, cc]) * w16
                    plsc.store_scatter(hr, [rows16, cc], v)
                return c2

            lax.fori_loop(0, D // 8, _cols, 0)

    def _block(bi, carry):
        pltpu.sync_copy(src_hbm.at[wid, pl.ds(bi * BLK, BLK)], srcb)
        pltpu.sync_copy(dst_hbm.at[wid, pl.ds(bi * BLK, BLK)], dstb)
        pltpu.sync_copy(ew_hbm.at[wid, pl.ds(bi * BLK, BLK)], ewb)
        _gather(0, hr_a)

        def _pair(p, c2):
            _wait_gather()

            @pl.when(p > 0)
            def _():
                _drain_scatter()

            _gather(2 * p + 1, hr_b)
            _compute(2 * p, hr_a)
            _scatter(2 * p, hr_a)
            _wait_gather()
            _compute(2 * p + 1, hr_b)
            _drain_scatter()

            @pl.when(p < BLK // 2 - 1)
            def _():
                _gather(2 * p + 2, hr_a)

            _scatter(2 * p + 1, hr_b)
            return c2

        lax.fori_loop(0, BLK // 2, _pair, 0)
        _drain_scatter()
        return carry

    lax.fori_loop(0, NBLK, _block, 0)

    plsc.subcore_barrier()
    for k in range(RPT // CHUNK):
        base = sid * RPT + k * CHUNK
        pltpu.sync_copy(acc.at[pl.ds(base, CHUNK)], hr_a)
        pltpu.sync_copy(hr_a, out_hbm.at[cid, pl.ds(base, CHUNK)])


_sc_agg = functools.partial(
    pl.kernel,
    out_type=jax.ShapeDtypeStruct((2, ACC_ROWS, WAUG), _f32),
    mesh=plsc.VectorSubcoreMesh(core_axis_name="c", subcore_axis_name="s"),
    compiler_params=pltpu.CompilerParams(
        needs_layout_passes=False, use_tc_tiling_on_sc=False),
    scratch_types=[
        pltpu.VMEM((NPAD,), _f32),
        pltpu.VMEM((BLK, CHUNK), _i32),
        pltpu.VMEM((BLK, CHUNK), _i32),
        pltpu.VMEM((BLK, CHUNK), _f32),
        pltpu.VMEM((CHUNK, WAUG), _f32),
        pltpu.VMEM((CHUNK, WAUG), _f32),
        pltpu.VMEM_SHARED((ACC_ROWS, WAUG), _f32),
        pltpu.SemaphoreType.DMA,
        pltpu.SemaphoreType.DMA,
    ],
)(_sc_body)


BN = 2000


def _emit_haug(h, as_row, ad_row, haug_ref, ald_ref):
    als = jnp.sum(h * as_row, axis=-1, keepdims=True)
    ald = jnp.sum(h * ad_row, axis=-1, keepdims=True)
    haug_ref[...] = jnp.concatenate(
        [h, als, jnp.zeros((h.shape[0], WAUG - D - 1), _f32)], axis=1)
    ald_ref[...] = ald


def _tc_pre_body(x_ref, w_ref, as_ref, ad_ref, haug_ref, ald_ref):
    h = jnp.dot(x_ref[...], w_ref[...], preferred_element_type=_f32)
    _emit_haug(h, as_ref[...], ad_ref[...], haug_ref, ald_ref)


def _norm_relu(p_ref, b_ref):
    u = p_ref[0, :, :D] + p_ref[1, :, :D]
    s = p_ref[0, :, D:D + 1] + p_ref[1, :, D:D + 1]
    return jnp.maximum(u / (s + 1e-16) + b_ref[...], 0.0)


def _tc_mid_body(p_ref, b_ref, w_ref, as_ref, ad_ref, haug_ref, ald_ref):
    g = _norm_relu(p_ref, b_ref)
    h = jnp.dot(g, w_ref[...], preferred_element_type=_f32)
    _emit_haug(h, as_ref[...], ad_ref[...], haug_ref, ald_ref)


def _tc_fin_body(p_ref, b_ref, o_ref):
    g = _norm_relu(p_ref, b_ref)
    m = jnp.max(g, axis=-1, keepdims=True)
    z = g - m
    lse = jnp.log(jnp.sum(jnp.exp(z), axis=-1, keepdims=True))
    o_ref[...] = z - lse


_vec_spec = pl.BlockSpec((1, D), lambda i: (0, 0))
_w_spec = pl.BlockSpec((D, D), lambda i: (0, 0))
_row_spec = pl.BlockSpec((BN, D), lambda i: (i, 0))
_aug_spec = pl.BlockSpec((BN, WAUG), lambda i: (i, 0))
_p_spec = pl.BlockSpec((2, BN, WAUG), lambda i: (0, i, 0))
_s_spec = pl.BlockSpec((BN, 1), lambda i: (i, 0))

_hout_shapes = (
    jax.ShapeDtypeStruct((N, WAUG), _f32),
    jax.ShapeDtypeStruct((N, 1), _f32),
)
_hout_specs = (_aug_spec, _s_spec)

_tc_pre = pl.pallas_call(
    _tc_pre_body,
    grid=(N // BN,),
    in_specs=[_row_spec, _w_spec, _vec_spec, _vec_spec],
    out_specs=_hout_specs,
    out_shape=_hout_shapes,
)

_tc_mid = pl.pallas_call(
    _tc_mid_body,
    grid=(N // BN,),
    in_specs=[_p_spec, _vec_spec, _w_spec, _vec_spec, _vec_spec],
    out_specs=_hout_specs,
    out_shape=_hout_shapes,
)

_tc_fin = pl.pallas_call(
    _tc_fin_body,
    grid=(N // BN,),
    in_specs=[_p_spec, _vec_spec],
    out_specs=_row_spec,
    out_shape=jax.ShapeDtypeStruct((N, D), _f32),
)


def kernel(x, edge_index, edge_weight, params):
    src = edge_index[0].astype(_i32)
    dst = edge_index[1].astype(_i32)
    ew = edge_weight.astype(_f32)
    pad = EPAD - E
    src_p = jnp.concatenate([src, jnp.zeros((pad,), _i32)]).reshape(NTILES, CPT, CHUNK)
    dst_p = jnp.concatenate([dst, jnp.full((pad,), DUMMY_ROW, _i32)]).reshape(NTILES, CPT, CHUNK)
    ew_p = jnp.concatenate([ew, jnp.zeros((pad,), _f32)]).reshape(NTILES, CPT, CHUNK)

    haug, ald = _tc_pre(x, params["W0"], params["as0"].reshape(1, D),
                        params["ad0"].reshape(1, D))
    for i in range(NL):
        ald_pad = jnp.pad(ald.reshape(N), (0, NPAD - N))
        part = _sc_agg(haug, ald_pad, src_p, dst_p, ew_p)
        b = params[f"b{i}"].reshape(1, D)
        if i + 1 < NL:
            haug, ald = _tc_mid(part, b, params[f"W{i + 1}"],
                                params[f"as{i + 1}"].reshape(1, D),
                                params[f"ad{i + 1}"].reshape(1, D))
        else:
            return _tc_fin(part, b)


# speedup vs baseline: 1.6082x; 1.6082x over previous
import functools

import jax
import jax.numpy as jnp
from jax import lax
from jax.experimental import pallas as pl
from jax.experimental.pallas import tpu as pltpu
from jax.experimental.pallas import tpu_sc as plsc

N = 10000
D = 128
E = 320000
NL = 6

NTILES = 32
CHUNK = 80
CPT = 128
EPT = CHUNK * CPT
EPAD = EPT * NTILES
WAUG = 144
ACC_ROWS = 10240
DUMMY_ROW = 10100
NPAD = ACC_ROWS
RPT = ACC_ROWS // 16
BLK = 16
NBLK = CPT // BLK
GP = CHUNK // 16

_f32 = jnp.float32
_i32 = jnp.int32


def _sc_body(h_hbm, ald_hbm, src_hbm, dst_hbm, ew_hbm, out_hbm,
             ald_v, srcb, dstb, ewb, hr_a, hr_b, wbuf, acc, gsem, ssem):
    cid = lax.axis_index("c")
    sid = lax.axis_index("s")
    wid = cid * 16 + sid

    pltpu.sync_copy(ald_hbm, ald_v)

    zeros16 = jnp.zeros((16,), _f32)
    for r in range(CHUNK):
        for k in range(WAUG // 16):
            hr_a[r, pl.ds(16 * k, 16)] = zeros16
    for k in range(RPT // CHUNK):
        pltpu.sync_copy(hr_a, acc.at[pl.ds(sid * RPT + k * CHUNK, CHUNK)])
    plsc.subcore_barrier()

    col128 = jnp.full((16,), D, _i32)
    lane = lax.iota(_i32, 16)

    def _gather(j, hr):
        pltpu.async_copy(h_hbm.at[srcb.at[j]], hr, gsem)

    def _wait_gather():
        pltpu.make_async_copy(h_hbm.at[srcb.at[0]], hr_a, gsem).wait()

    def _scatter(j, hr):
        pltpu.async_copy(hr, acc.at[dstb.at[j]], ssem, add=True)

    def _drain_scatter():
        pltpu.make_async_copy(hr_a, acc.at[dstb.at[0]], ssem).wait()

    def _compute(j, hr):
        for g in range(GP):
            rows16 = 16 * g + lane
            d16 = dstb[j, pl.ds(16 * g, 16)]
            als16 = plsc.load_gather(hr, [rows16, col128])
            ald16 = plsc.load_gather(ald_v, [d16])
            a16 = als16 + ald16
            e16 = jnp.maximum(a16, 0.2 * a16)
            ex16 = jnp.exp(e16)
            w16 = ex16 * ewb[j, pl.ds(16 * g, 16)]
            wbuf[pl.ds(16 * g, 16)] = w16
            plsc.store_scatter(hr, [rows16, col128], ex16)
        for e in range(CHUNK):
            wv = plsc.load_gather(wbuf, [jnp.full((16,), e, _i32)])
            for k in range(D // 16):
                hr[e, pl.ds(16 * k, 16)] = hr[e, pl.ds(16 * k, 16)] * wv

    def _block(bi, carry):
        pltpu.sync_copy(src_hbm.at[wid, pl.ds(bi * BLK, BLK)], srcb)
        pltpu.sync_copy(dst_hbm.at[wid, pl.ds(bi * BLK, BLK)], dstb)
        pltpu.sync_copy(ew_hbm.at[wid, pl.ds(bi * BLK, BLK)], ewb)
        _gather(0, hr_a)

        def _pair(p, c2):
            _wait_gather()

            @pl.when(p > 0)
            def _():
                _drain_scatter()

            _gather(2 * p + 1, hr_b)
            _compute(2 * p, hr_a)
            _scatter(2 * p, hr_a)
            _wait_gather()
            _compute(2 * p + 1, hr_b)
            _drain_scatter()

            @pl.when(p < BLK // 2 - 1)
            def _():
                _gather(2 * p + 2, hr_a)

            _scatter(2 * p + 1, hr_b)
            return c2

        lax.fori_loop(0, BLK // 2, _pair, 0)
        _drain_scatter()
        return carry

    lax.fori_loop(0, NBLK, _block, 0)

    plsc.subcore_barrier()
    for k in range(RPT // CHUNK):
        base = sid * RPT + k * CHUNK
        pltpu.sync_copy(acc.at[pl.ds(base, CHUNK)], hr_a)
        pltpu.sync_copy(hr_a, out_hbm.at[cid, pl.ds(base, CHUNK)])


_sc_agg = functools.partial(
    pl.kernel,
    out_type=jax.ShapeDtypeStruct((2, ACC_ROWS, WAUG), _f32),
    mesh=plsc.VectorSubcoreMesh(core_axis_name="c", subcore_axis_name="s"),
    compiler_params=pltpu.CompilerParams(
        needs_layout_passes=False, use_tc_tiling_on_sc=False),
    scratch_types=[
        pltpu.VMEM((NPAD,), _f32),
        pltpu.VMEM((BLK, CHUNK), _i32),
        pltpu.VMEM((BLK, CHUNK), _i32),
        pltpu.VMEM((BLK, CHUNK), _f32),
        pltpu.VMEM((CHUNK, WAUG), _f32),
        pltpu.VMEM((CHUNK, WAUG), _f32),
        pltpu.VMEM((CHUNK,), _f32),
        pltpu.VMEM_SHARED((ACC_ROWS, WAUG), _f32),
        pltpu.SemaphoreType.DMA,
        pltpu.SemaphoreType.DMA,
    ],
)(_sc_body)


BN = 2000


def _emit_haug(h, as_row, ad_row, haug_ref, ald_ref):
    als = jnp.sum(h * as_row, axis=-1, keepdims=True)
    ald = jnp.sum(h * ad_row, axis=-1, keepdims=True)
    haug_ref[...] = jnp.concatenate(
        [h, als, jnp.zeros((h.shape[0], WAUG - D - 1), _f32)], axis=1)
    ald_ref[...] = ald


def _tc_pre_body(x_ref, w_ref, as_ref, ad_ref, haug_ref, ald_ref):
    h = jnp.dot(x_ref[...], w_ref[...], preferred_element_type=_f32)
    _emit_haug(h, as_ref[...], ad_ref[...], haug_ref, ald_ref)


def _norm_relu(p_ref, b_ref):
    u = p_ref[0, :, :D] + p_ref[1, :, :D]
    s = p_ref[0, :, D:D + 1] + p_ref[1, :, D:D + 1]
    return jnp.maximum(u / (s + 1e-16) + b_ref[...], 0.0)


def _tc_mid_body(p_ref, b_ref, w_ref, as_ref, ad_ref, haug_ref, ald_ref):
    g = _norm_relu(p_ref, b_ref)
    h = jnp.dot(g, w_ref[...], preferred_element_type=_f32)
    _emit_haug(h, as_ref[...], ad_ref[...], haug_ref, ald_ref)


def _tc_fin_body(p_ref, b_ref, o_ref):
    g = _norm_relu(p_ref, b_ref)
    m = jnp.max(g, axis=-1, keepdims=True)
    z = g - m
    lse = jnp.log(jnp.sum(jnp.exp(z), axis=-1, keepdims=True))
    o_ref[...] = z - lse


_vec_spec = pl.BlockSpec((1, D), lambda i: (0, 0))
_w_spec = pl.BlockSpec((D, D), lambda i: (0, 0))
_row_spec = pl.BlockSpec((BN, D), lambda i: (i, 0))
_aug_spec = pl.BlockSpec((BN, WAUG), lambda i: (i, 0))
_p_spec = pl.BlockSpec((2, BN, WAUG), lambda i: (0, i, 0))
_s_spec = pl.BlockSpec((BN, 1), lambda i: (i, 0))

_hout_shapes = (
    jax.ShapeDtypeStruct((N, WAUG), _f32),
    jax.ShapeDtypeStruct((N, 1), _f32),
)
_hout_specs = (_aug_spec, _s_spec)

_tc_pre = pl.pallas_call(
    _tc_pre_body,
    grid=(N // BN,),
    in_specs=[_row_spec, _w_spec, _vec_spec, _vec_spec],
    out_specs=_hout_specs,
    out_shape=_hout_shapes,
)

_tc_mid = pl.pallas_call(
    _tc_mid_body,
    grid=(N // BN,),
    in_specs=[_p_spec, _vec_spec, _w_spec, _vec_spec, _vec_spec],
    out_specs=_hout_specs,
    out_shape=_hout_shapes,
)

_tc_fin = pl.pallas_call(
    _tc_fin_body,
    grid=(N // BN,),
    in_specs=[_p_spec, _vec_spec],
    out_specs=_row_spec,
    out_shape=jax.ShapeDtypeStruct((N, D), _f32),
)


def kernel(x, edge_index, edge_weight, params):
    src = edge_index[0].astype(_i32)
    dst = edge_index[1].astype(_i32)
    ew = edge_weight.astype(_f32)
    pad = EPAD - E
    src_p = jnp.concatenate([src, jnp.zeros((pad,), _i32)]).reshape(NTILES, CPT, CHUNK)
    dst_p = jnp.concatenate([dst, jnp.full((pad,), DUMMY_ROW, _i32)]).reshape(NTILES, CPT, CHUNK)
    ew_p = jnp.concatenate([ew, jnp.zeros((pad,), _f32)]).reshape(NTILES, CPT, CHUNK)

    haug, ald = _tc_pre(x, params["W0"], params["as0"].reshape(1, D),
                        params["ad0"].reshape(1, D))
    for i in range(NL):
        ald_pad = jnp.pad(ald.reshape(N), (0, NPAD - N))
        part = _sc_agg(haug, ald_pad, src_p, dst_p, ew_p)
        b = params[f"b{i}"].reshape(1, D)
        if i + 1 < NL:
            haug, ald = _tc_mid(part, b, params[f"W{i + 1}"],
                                params[f"as{i + 1}"].reshape(1, D),
                                params[f"ad{i + 1}"].reshape(1, D))
        else:
            return _tc_fin(part, b)


# speedup vs baseline: 1.6241x; 1.0099x over previous
import functools

import jax
import jax.numpy as jnp
from jax import lax
from jax.experimental import pallas as pl
from jax.experimental.pallas import tpu as pltpu
from jax.experimental.pallas import tpu_sc as plsc

N = 10000
D = 128
E = 320000
NL = 6

NTILES = 32
CHUNK = 80
CPT = 128
EPT = CHUNK * CPT
EPAD = EPT * NTILES
WAUG = 144
ACC_ROWS = 10240
DUMMY_ROW = 10100
NPAD = ACC_ROWS
RPT = ACC_ROWS // 16
BLK = 16
NBLK = CPT // BLK
GP = CHUNK // 16

_f32 = jnp.float32
_i32 = jnp.int32


def _sc_body(h_hbm, ald_hbm, src_hbm, dst_hbm, ew_hbm, out_hbm,
             ald_v, srcb, dstb, ewb, hr_a, hr_b, wbuf, acc, gsem, ssem):
    cid = lax.axis_index("c")
    sid = lax.axis_index("s")
    wid = cid * 16 + sid

    pltpu.async_copy(ald_hbm, ald_v, gsem)

    zeros16 = jnp.zeros((16,), _f32)
    for r in range(CHUNK):
        for k in range(WAUG // 16):
            hr_a[r, pl.ds(16 * k, 16)] = zeros16
    for k in range(RPT // CHUNK):
        pltpu.async_copy(hr_a, acc.at[pl.ds(sid * RPT + k * CHUNK, CHUNK)], ssem)
    pltpu.make_async_copy(ald_hbm, ald_v, gsem).wait()
    for k in range(RPT // CHUNK):
        pltpu.make_async_copy(hr_a, acc.at[pl.ds(sid * RPT, CHUNK)], ssem).wait()
    plsc.subcore_barrier()

    col128 = jnp.full((16,), D, _i32)
    lane = lax.iota(_i32, 16)

    def _gather(j, hr):
        pltpu.async_copy(h_hbm.at[srcb.at[j]], hr, gsem)

    def _wait_gather():
        pltpu.make_async_copy(h_hbm.at[srcb.at[0]], hr_a, gsem).wait()

    def _scatter(j, hr):
        pltpu.async_copy(hr, acc.at[dstb.at[j]], ssem, add=True)

    def _drain_scatter():
        pltpu.make_async_copy(hr_a, acc.at[dstb.at[0]], ssem).wait()

    def _compute(j, hr):
        for g in range(GP):
            rows16 = 16 * g + lane
            d16 = dstb[j, pl.ds(16 * g, 16)]
            als16 = plsc.load_gather(hr, [rows16, col128])
            ald16 = plsc.load_gather(ald_v, [d16])
            a16 = als16 + ald16
            e16 = jnp.maximum(a16, 0.2 * a16)
            ex16 = jnp.exp(e16)
            w16 = ex16 * ewb[j, pl.ds(16 * g, 16)]
            wbuf[pl.ds(16 * g, 16)] = w16
            plsc.store_scatter(hr, [rows16, col128], ex16)
        for e in range(CHUNK):
            wv = plsc.load_gather(wbuf, [jnp.full((16,), e, _i32)])
            for k in range(D // 16):
                hr[e, pl.ds(16 * k, 16)] = hr[e, pl.ds(16 * k, 16)] * wv

    def _block(bi, carry):
        pltpu.sync_copy(src_hbm.at[wid, pl.ds(bi * BLK, BLK)], srcb)
        pltpu.sync_copy(dst_hbm.at[wid, pl.ds(bi * BLK, BLK)], dstb)
        pltpu.sync_copy(ew_hbm.at[wid, pl.ds(bi * BLK, BLK)], ewb)
        _gather(0, hr_a)

        def _pair(p, c2):
            _wait_gather()

            @pl.when(p > 0)
            def _():
                _drain_scatter()

            _gather(2 * p + 1, hr_b)
            _compute(2 * p, hr_a)
            _scatter(2 * p, hr_a)
            _wait_gather()
            _compute(2 * p + 1, hr_b)
            _drain_scatter()

            @pl.when(p < BLK // 2 - 1)
            def _():
                _gather(2 * p + 2, hr_a)

            _scatter(2 * p + 1, hr_b)
            return c2

        lax.fori_loop(0, BLK // 2, _pair, 0)
        _drain_scatter()
        return carry

    lax.fori_loop(0, NBLK, _block, 0)

    plsc.subcore_barrier()
    bufs = (hr_a, hr_b)
    nd = RPT // CHUNK

    def _acc_slice(k):
        return acc.at[pl.ds(sid * RPT + k * CHUNK, CHUNK)]

    pltpu.async_copy(_acc_slice(0), hr_a, gsem)
    for k in range(nd):
        cur, nxt = bufs[k % 2], bufs[(k + 1) % 2]
        pltpu.make_async_copy(_acc_slice(0), cur, gsem).wait()
        if k >= 1:
            pltpu.make_async_copy(nxt, out_hbm.at[cid, pl.ds(0, CHUNK)],
                                  ssem).wait()
        if k + 1 < nd:
            pltpu.async_copy(_acc_slice(k + 1), nxt, gsem)
        pltpu.async_copy(cur, out_hbm.at[cid, pl.ds(sid * RPT + k * CHUNK, CHUNK)],
                         ssem)
    pltpu.make_async_copy(bufs[(nd - 1) % 2],
                          out_hbm.at[cid, pl.ds(0, CHUNK)], ssem).wait()


_sc_agg = functools.partial(
    pl.kernel,
    out_type=jax.ShapeDtypeStruct((2, ACC_ROWS, WAUG), _f32),
    mesh=plsc.VectorSubcoreMesh(core_axis_name="c", subcore_axis_name="s"),
    compiler_params=pltpu.CompilerParams(
        needs_layout_passes=False, use_tc_tiling_on_sc=False),
    scratch_types=[
        pltpu.VMEM((NPAD,), _f32),
        pltpu.VMEM((BLK, CHUNK), _i32),
        pltpu.VMEM((BLK, CHUNK), _i32),
        pltpu.VMEM((BLK, CHUNK), _f32),
        pltpu.VMEM((CHUNK, WAUG), _f32),
        pltpu.VMEM((CHUNK, WAUG), _f32),
        pltpu.VMEM((CHUNK,), _f32),
        pltpu.VMEM_SHARED((ACC_ROWS, WAUG), _f32),
        pltpu.SemaphoreType.DMA,
        pltpu.SemaphoreType.DMA,
    ],
)(_sc_body)


BN = 2000


def _emit_haug(h, as_row, ad_row, haug_ref, ald_ref):
    als = jnp.sum(h * as_row, axis=-1, keepdims=True)
    ald = jnp.sum(h * ad_row, axis=-1, keepdims=True)
    haug_ref[...] = jnp.concatenate(
        [h, als, jnp.zeros((h.shape[0], WAUG - D - 1), _f32)], axis=1)
    ald_ref[...] = ald


def _tc_pre_body(x_ref, w_ref, as_ref, ad_ref, haug_ref, ald_ref):
    h = jnp.dot(x_ref[...], w_ref[...], preferred_element_type=_f32)
    _emit_haug(h, as_ref[...], ad_ref[...], haug_ref, ald_ref)


def _norm_relu(p_ref, b_ref):
    u = p_ref[0, :, :D] + p_ref[1, :, :D]
    s = p_ref[0, :, D:D + 1] + p_ref[1, :, D:D + 1]
    return jnp.maximum(u / (s + 1e-16) + b_ref[...], 0.0)


def _tc_mid_body(p_ref, b_ref, w_ref, as_ref, ad_ref, haug_ref, ald_ref):
    g = _norm_relu(p_ref, b_ref)
    h = jnp.dot(g, w_ref[...], preferred_element_type=_f32)
    _emit_haug(h, as_ref[...], ad_ref[...], haug_ref, ald_ref)


def _tc_fin_body(p_ref, b_ref, o_ref):
    g = _norm_relu(p_ref, b_ref)
    m = jnp.max(g, axis=-1, keepdims=True)
    z = g - m
    lse = jnp.log(jnp.sum(jnp.exp(z), axis=-1, keepdims=True))
    o_ref[...] = z - lse


_vec_spec = pl.BlockSpec((1, D), lambda i: (0, 0))
_w_spec = pl.BlockSpec((D, D), lambda i: (0, 0))
_row_spec = pl.BlockSpec((BN, D), lambda i: (i, 0))
_aug_spec = pl.BlockSpec((BN, WAUG), lambda i: (i, 0))
_p_spec = pl.BlockSpec((2, BN, WAUG), lambda i: (0, i, 0))
_s_spec = pl.BlockSpec((BN, 1), lambda i: (i, 0))

_hout_shapes = (
    jax.ShapeDtypeStruct((N, WAUG), _f32),
    jax.ShapeDtypeStruct((N, 1), _f32),
)
_hout_specs = (_aug_spec, _s_spec)

_tc_pre = pl.pallas_call(
    _tc_pre_body,
    grid=(N // BN,),
    in_specs=[_row_spec, _w_spec, _vec_spec, _vec_spec],
    out_specs=_hout_specs,
    out_shape=_hout_shapes,
)

_tc_mid = pl.pallas_call(
    _tc_mid_body,
    grid=(N // BN,),
    in_specs=[_p_spec, _vec_spec, _w_spec, _vec_spec, _vec_spec],
    out_specs=_hout_specs,
    out_shape=_hout_shapes,
)

_tc_fin = pl.pallas_call(
    _tc_fin_body,
    grid=(N // BN,),
    in_specs=[_p_spec, _vec_spec],
    out_specs=_row_spec,
    out_shape=jax.ShapeDtypeStruct((N, D), _f32),
)


def kernel(x, edge_index, edge_weight, params):
    src = edge_index[0].astype(_i32)
    dst = edge_index[1].astype(_i32)
    ew = edge_weight.astype(_f32)
    pad = EPAD - E
    src_p = jnp.concatenate([src, jnp.zeros((pad,), _i32)]).reshape(NTILES, CPT, CHUNK)
    dst_p = jnp.concatenate([dst, jnp.full((pad,), DUMMY_ROW, _i32)]).reshape(NTILES, CPT, CHUNK)
    ew_p = jnp.concatenate([ew, jnp.zeros((pad,), _f32)]).reshape(NTILES, CPT, CHUNK)

    haug, ald = _tc_pre(x, params["W0"], params["as0"].reshape(1, D),
                        params["ad0"].reshape(1, D))
    for i in range(NL):
        ald_pad = jnp.pad(ald.reshape(N), (0, NPAD - N))
        part = _sc_agg(haug, ald_pad, src_p, dst_p, ew_p)
        b = params[f"b{i}"].reshape(1, D)
        if i + 1 < NL:
            haug, ald = _tc_mid(part, b, params[f"W{i + 1}"],
                                params[f"as{i + 1}"].reshape(1, D),
                                params[f"ad{i + 1}"].reshape(1, D))
        else:
            return _tc_fin(part, b)
